# baseline (speedup 1.0000x reference)
import jax
import jax.numpy as jnp
from jax import lax
from jax.experimental import pallas as pl
from jax.experimental.pallas import tpu as pltpu

N = 2048
DIM = 768
H = 12
KD = 32
D = 64
QKV = H * KD * 2 + H * D
CBS = 16
CST = 8
TOPN = 16
WIN = 512
KC = H * KD
VC = H * D
M = (N - CBS) // CST + 1
MP = 256
QT = 256
NT = N // QT
SCALE = KD ** (-0.5)
SCALE_W = (DIM // H) ** (-0.5)
F32 = jnp.float32
BF16 = jnp.bfloat16
NEG = -1e30


def _gelu(x):
    return 0.5 * x * (1.0 + lax.erf(x * 0.7071067811865476))


def _sigmoid(x):
    return 1.0 / (1.0 + jnp.exp(-x))


def _dot_nt(a, b):
    return lax.dot_general(a.astype(BF16), b.astype(BF16),
                           (((1,), (1,)), ((), ())),
                           preferred_element_type=F32)


def _dotb(a, b):
    return jnp.dot(a.astype(BF16), b.astype(BF16),
                   preferred_element_type=F32)


def _proj_body(x_ref, w_ref, b_ref, o_ref):
    o_ref[...] = _dotb(x_ref[...], w_ref[...]) + b_ref[...]


def _blk_body(seg, kf_ref, pos_ref, o_ref):
    for i in range(CBS):
        li = kf_ref[(i // CST):(i // CST) + MP,
                    (i % CST) * seg:((i % CST) + 1) * seg]
        o_ref[:, i * seg:(i + 1) * seg] = (
            li + pos_ref[:, i * seg:(i + 1) * seg]).astype(BF16)


def _cmp1_body(bl_ref, w1_ref, b1_ref, o_ref):
    o_ref[...] = _gelu(jnp.dot(bl_ref[...], w1_ref[...],
                               preferred_element_type=F32) + b1_ref[...])


def _cmp2_body(h_ref, w2_ref, b2_ref, o_ref):
    o_ref[...] = _dotb(h_ref[...], w2_ref[...]) + b2_ref[...]


def _catt_body(qc_ref, qs_ref, ck_ref, cv_ref, ocmp_ref, obm_ref):
    maddrow = jnp.where(
        lax.broadcasted_iota(jnp.int32, (1, MP), 1) < M, 0.0, NEG)
    imp = jnp.zeros((QT, MP), F32)
    for h in range(H):
        ckh = ck_ref[:, h * KD:(h + 1) * KD]
        s = _dot_nt(qc_ref[:, h * KD:(h + 1) * KD], ckh) * SCALE + maddrow
        mm = jnp.max(s, axis=1, keepdims=True)
        el = jnp.exp(s - mm)
        inv = 1.0 / jnp.sum(el, axis=1, keepdims=True)
        ocmp_ref[:, h * D:(h + 1) * D] = _dotb(
            el, cv_ref[:, h * D:(h + 1) * D]) * inv
        s2 = _dot_nt(qs_ref[:, h * KD:(h + 1) * KD], ckh) * SCALE + maddrow
        mm2 = jnp.max(s2, axis=1, keepdims=True)
        el2 = jnp.exp(s2 - mm2)
        inv2 = 1.0 / jnp.sum(el2, axis=1, keepdims=True)
        imp = imp + el2 * inv2
    impm = imp + maddrow
    vals = impm
    for _ in range(TOPN - 1):
        mx = jnp.max(vals, axis=1, keepdims=True)
        vals = jnp.where(vals >= mx, NEG, vals)
    thr = jnp.max(vals, axis=1, keepdims=True)
    shifted = jnp.concatenate(
        [jnp.full((QT, 1), NEG, F32), impm[:, :MP - 1]], axis=1)
    impm2 = jnp.maximum(impm, shifted)
    obm_ref[...] = (impm2 >= thr).astype(F32)


def _slc_body(bm_ref, q_ref, k_ref, v_ref, o_ref,
              madd_scr, sc0, sc1, pb0, pb1, mx0, mx1, l0, l1):
    qt = pl.program_id(0)
    rows = qt * QT + lax.broadcasted_iota(jnp.int32, (QT, QT), 0)
    prow = lax.broadcasted_iota(jnp.int32, (32, QT), 0)
    jcol = lax.broadcasted_iota(jnp.int32, (32, QT), 1)
    e = (jcol // 8 == prow).astype(F32)
    for kt in range(NT):
        @pl.when(kt <= qt)
        def _mk(kt=kt):
            km = _dotb(bm_ref[:, kt * 32:(kt + 1) * 32], e)
            cols = kt * QT + lax.broadcasted_iota(jnp.int32, (QT, QT), 1)
            keep = (cols <= rows) & (km > 0.5)
            madd_scr[:, kt * QT:(kt + 1) * QT] = jnp.where(keep, 0.0, NEG)

        @pl.when(kt > qt)
        def _z(kt=kt):
            zz = jnp.zeros((QT, QT), BF16)
            pb0[:, kt * QT:(kt + 1) * QT] = zz
            pb1[:, kt * QT:(kt + 1) * QT] = zz

    for h in range(H):
        sc = sc0 if h % 2 == 0 else sc1
        pb = pb0 if h % 2 == 0 else pb1
        mx = mx0 if h % 2 == 0 else mx1
        lv = l0 if h % 2 == 0 else l1
        q = q_ref[:, h * KD:(h + 1) * KD]
        mx[...] = jnp.full_like(mx, NEG)
        for kt in range(NT):
            @pl.when(kt <= qt)
            def _qk(kt=kt, q=q, sc=sc, mx=mx):
                k = k_ref[kt * QT:(kt + 1) * QT, h * KD:(h + 1) * KD]
                s = (_dot_nt(q, k) * SCALE
                     + madd_scr[:, kt * QT:(kt + 1) * QT])
                sc[:, kt * QT:(kt + 1) * QT] = s
                mx[:, kt:kt + 1] = jnp.max(s, axis=1, keepdims=True)
        m = jnp.max(mx[...], axis=1, keepdims=True)
        lv[...] = jnp.zeros_like(lv)
        for kt in range(NT):
            @pl.when(kt <= qt)
            def _ex(kt=kt, m=m, sc=sc, pb=pb, lv=lv):
                el = jnp.exp(sc[:, kt * QT:(kt + 1) * QT] - m)
                pb[:, kt * QT:(kt + 1) * QT] = el.astype(BF16)
                lv[...] = lv[...] + jnp.sum(el, axis=1, keepdims=True)
        inv = jnp.where(m > -1e29, 1.0 / lv[...], 0.0)
        pv = jnp.dot(pb[...], v_ref[:, h * D:(h + 1) * D].astype(BF16),
                     preferred_element_type=F32)
        o_ref[:, h * D:(h + 1) * D] = pv * inv


def _win_body(q_ref, k_ref, v_ref, o_ref, madd_scr, sc0, sc1, pb0, pb1):
    qt = pl.program_id(0)
    rows = qt * QT + lax.broadcasted_iota(jnp.int32, (QT, QT), 0)
    for dd in range(3):
        kt = qt - 2 + dd
        cols = kt * QT + lax.broadcasted_iota(jnp.int32, (QT, QT), 1)
        keep = (cols <= rows) & (cols > rows - WIN) & (kt >= 0)
        madd_scr[:, dd * QT:(dd + 1) * QT] = jnp.where(keep, 0.0, NEG)
    for h in range(H):
        sc = sc0 if h % 2 == 0 else sc1
        pb = pb0 if h % 2 == 0 else pb1
        q = q_ref[:, h * KD:(h + 1) * KD]
        mxs = []
        for dd in range(3):
            kt = jnp.maximum(qt - 2 + dd, 0)
            k = k_ref[pl.ds(kt * QT, QT), h * KD:(h + 1) * KD]
            s = _dot_nt(q, k) * SCALE_W + madd_scr[:, dd * QT:(dd + 1) * QT]
            sc[:, dd * QT:(dd + 1) * QT] = s
            mxs.append(jnp.max(s, axis=1, keepdims=True))
        m = jnp.maximum(jnp.maximum(mxs[0], mxs[1]), mxs[2])
        l = jnp.zeros((QT, 1), F32)
        for dd in range(3):
            el = jnp.exp(sc[:, dd * QT:(dd + 1) * QT] - m)
            pb[:, dd * QT:(dd + 1) * QT] = el.astype(BF16)
            l = l + jnp.sum(el, axis=1, keepdims=True)
        inv = 1.0 / l
        acc = jnp.zeros((QT, D), F32)
        for dd in range(3):
            kt = jnp.maximum(qt - 2 + dd, 0)
            v = v_ref[pl.ds(kt * QT, QT), h * D:(h + 1) * D]
            acc = acc + jnp.dot(pb[:, dd * QT:(dd + 1) * QT],
                                v.astype(BF16), preferred_element_type=F32)
        o_ref[:, h * D:(h + 1) * D] = acc * inv


def _fin_body(x_ref, gw1_ref, gb1_ref, gw2_ref, gb2_ref,
              cmp_ref, slc_ref, win_ref, pw_ref, pb_ref, o_ref):
    gh = _gelu(_dotb(x_ref[...], gw1_ref[...]) + gb1_ref[...])
    g = _sigmoid(_dotb(gh, gw2_ref[...]) + gb2_ref[...])
    comb = (g[:, 0:1] * cmp_ref[...] + g[:, 1:2] * slc_ref[...]
            + g[:, 2:3] * win_ref[...])
    o_ref[...] = _dotb(comb, pw_ref[...]) + pb_ref[...]


def _full(shape, imap):
    return pl.BlockSpec(shape, imap)


def kernel(x, W_cmp, b_cmp, W_slc, b_slc, W_win, b_win,
           k_pos, k_W1, k_b1, k_W2, k_b2,
           v_pos, v_W1, v_b1, v_W2, v_b2,
           g_W1, g_b1, g_W2, g_b2, p_W, p_b):
    x2 = x[0]
    Wall = jnp.concatenate([W_cmp, W_slc, W_win], axis=1).astype(BF16)
    ball = jnp.concatenate([b_cmp, b_slc, b_win])[None, :]

    qkv = pl.pallas_call(
        _proj_body,
        grid=(6,),
        in_specs=[
            _full((N, DIM), lambda ct: (0, 0)),
            _full((DIM, 768), lambda ct: (0, ct)),
            _full((1, 768), lambda ct: (0, ct)),
        ],
        out_specs=_full((N, 768), lambda ct: (0, ct)),
        out_shape=jax.ShapeDtypeStruct((N, 3 * QKV), F32),
    )(x2, Wall, ball)

    kflat = qkv[:, KC:2 * KC]
    vflat = qkv[:, 2 * KC:QKV]
    kfr = jnp.pad(kflat, ((0, 8), (0, 0))).reshape(MP + 1, 8 * KC)
    vfr = jnp.pad(vflat, ((0, 8), (0, 0))).reshape(MP + 1, 8 * VC)
    kposf = k_pos.reshape(1, CBS * KC)
    vposf = v_pos.reshape(1, CBS * VC)

    kbl = pl.pallas_call(
        lambda *a: _blk_body(KC, *a),
        grid=(1,),
        in_specs=[
            _full((MP + 1, 8 * KC), lambda i: (0, 0)),
            _full((1, CBS * KC), lambda i: (0, 0)),
        ],
        out_specs=_full((MP, CBS * KC), lambda i: (0, 0)),
        out_shape=jax.ShapeDtypeStruct((MP, CBS * KC), BF16),
    )(kfr, kposf)

    vbl = pl.pallas_call(
        lambda *a: _blk_body(VC, *a),
        grid=(1,),
        in_specs=[
            _full((MP + 1, 8 * VC), lambda i: (0, 0)),
            _full((1, CBS * VC), lambda i: (0, 0)),
        ],
        out_specs=_full((MP, CBS * VC), lambda i: (0, 0)),
        out_shape=jax.ShapeDtypeStruct((MP, CBS * VC), BF16),
    )(vfr, vposf)

    hk = pl.pallas_call(
        _cmp1_body,
        grid=(1,),
        in_specs=[
            _full((MP, CBS * KC), lambda i: (0, 0)),
            _full((CBS * KC, 2 * KC), lambda i: (0, 0)),
            _full((1, 2 * KC), lambda i: (0, 0)),
        ],
        out_specs=_full((MP, 2 * KC), lambda i: (0, 0)),
        out_shape=jax.ShapeDtypeStruct((MP, 2 * KC), F32),
    )(kbl, k_W1.astype(BF16), k_b1[None, :])

    ck = pl.pallas_call(
        _cmp2_body,
        grid=(1,),
        in_specs=[
            _full((MP, 2 * KC), lambda i: (0, 0)),
            _full((2 * KC, KC), lambda i: (0, 0)),
            _full((1, KC), lambda i: (0, 0)),
        ],
        out_specs=_full((MP, KC), lambda i: (0, 0)),
        out_shape=jax.ShapeDtypeStruct((MP, KC), F32),
    )(hk, k_W2.astype(BF16), k_b2[None, :])

    hv = pl.pallas_call(
        _cmp1_body,
        grid=(3,),
        in_specs=[
            _full((MP, CBS * VC), lambda ct: (0, 0)),
            _full((CBS * VC, 512), lambda ct: (0, ct)),
            _full((1, 512), lambda ct: (0, ct)),
        ],
        out_specs=_full((MP, 512), lambda ct: (0, ct)),
        out_shape=jax.ShapeDtypeStruct((MP, 2 * VC), F32),
    )(vbl, v_W1.astype(BF16), v_b1[None, :])

    cv = pl.pallas_call(
        _cmp2_body,
        grid=(1,),
        in_specs=[
            _full((MP, 2 * VC), lambda i: (0, 0)),
            _full((2 * VC, VC), lambda i: (0, 0)),
            _full((1, VC), lambda i: (0, 0)),
        ],
        out_specs=_full((MP, VC), lambda i: (0, 0)),
        out_shape=jax.ShapeDtypeStruct((MP, VC), F32),
    )(hv, v_W2.astype(BF16), v_b2[None, :])

    out_cmp, bmask = pl.pallas_call(
        _catt_body,
        grid=(NT,),
        in_specs=[
            _full((QT, KC), lambda qt: (qt, 0)),
            _full((QT, KC), lambda qt: (qt, 4)),
            _full((MP, KC), lambda qt: (0, 0)),
            _full((MP, VC), lambda qt: (0, 0)),
        ],
        out_specs=[
            _full((QT, VC), lambda qt: (qt, 0)),
            _full((QT, MP), lambda qt: (qt, 0)),
        ],
        out_shape=[
            jax.ShapeDtypeStruct((N, VC), F32),
            jax.ShapeDtypeStruct((N, MP), F32),
        ],
    )(qkv, qkv, ck, cv)

    out_slc = pl.pallas_call(
        _slc_body,
        grid=(NT,),
        in_specs=[
            _full((QT, MP), lambda qt: (qt, 0)),
            _full((QT, KC), lambda qt: (qt, 4)),
            _full((N, KC), lambda qt: (0, 5)),
            _full((N, VC), lambda qt: (0, 3)),
        ],
        out_specs=_full((QT, VC), lambda qt: (qt, 0)),
        out_shape=jax.ShapeDtypeStruct((N, VC), F32),
        scratch_shapes=[
            pltpu.VMEM((QT, N), F32),
            pltpu.VMEM((QT, N), F32),
            pltpu.VMEM((QT, N), F32),
            pltpu.VMEM((QT, N), BF16),
            pltpu.VMEM((QT, N), BF16),
            pltpu.VMEM((QT, NT), F32),
            pltpu.VMEM((QT, NT), F32),
            pltpu.VMEM((QT, 1), F32),
            pltpu.VMEM((QT, 1), F32),
        ],
    )(bmask, qkv, qkv, qkv)

    out_win = pl.pallas_call(
        _win_body,
        grid=(NT,),
        in_specs=[
            _full((QT, KC), lambda qt: (qt, 8)),
            _full((N, KC), lambda qt: (0, 9)),
            _full((N, VC), lambda qt: (0, 5)),
        ],
        out_specs=_full((QT, VC), lambda qt: (qt, 0)),
        out_shape=jax.ShapeDtypeStruct((N, VC), F32),
        scratch_shapes=[
            pltpu.VMEM((QT, 3 * QT), F32),
            pltpu.VMEM((QT, 3 * QT), F32),
            pltpu.VMEM((QT, 3 * QT), F32),
            pltpu.VMEM((QT, 3 * QT), BF16),
            pltpu.VMEM((QT, 3 * QT), BF16),
        ],
    )(qkv, qkv, qkv)

    gW2p = jnp.pad(g_W2, ((0, 0), (0, 125)))
    gb2p = jnp.pad(g_b2, (0, 125))[None, :]
    out = pl.pallas_call(
        _fin_body,
        grid=(NT,),
        in_specs=[
            _full((QT, DIM), lambda qt: (qt, 0)),
            _full((DIM, DIM // 2), lambda qt: (0, 0)),
            _full((1, DIM // 2), lambda qt: (0, 0)),
            _full((DIM // 2, 128), lambda qt: (0, 0)),
            _full((1, 128), lambda qt: (0, 0)),
            _full((QT, VC), lambda qt: (qt, 0)),
            _full((QT, VC), lambda qt: (qt, 0)),
            _full((QT, VC), lambda qt: (qt, 0)),
            _full((VC, DIM), lambda qt: (0, 0)),
            _full((1, DIM), lambda qt: (0, 0)),
        ],
        out_specs=_full((QT, DIM), lambda qt: (qt, 0)),
        out_shape=jax.ShapeDtypeStruct((N, DIM), F32),
    )(x2, g_W1.astype(BF16), g_b1[None, :], gW2p.astype(BF16), gb2p,
      out_cmp, out_slc, out_win, p_W.astype(BF16), p_b[None, :])

    return out[None, :, :]


# speedup vs baseline: 1.2001x; 1.2001x over previous
import jax
import jax.numpy as jnp
from jax import lax
from jax.experimental import pallas as pl
from jax.experimental.pallas import tpu as pltpu

N = 2048
DIM = 768
H = 12
KD = 32
D = 64
QKV = H * KD * 2 + H * D
CBS = 16
CST = 8
TOPN = 16
WIN = 512
KC = H * KD
VC = H * D
M = (N - CBS) // CST + 1
MP = 256
QT = 256
NT = N // QT
SCALE = KD ** (-0.5)
SCALE_W = (DIM // H) ** (-0.5)
F32 = jnp.float32
BF16 = jnp.bfloat16
NEG = -1e30


def _gelu(x):
    return 0.5 * x * (1.0 + lax.erf(x * 0.7071067811865476))


def _sigmoid(x):
    return 1.0 / (1.0 + jnp.exp(-x))


def _dot_nt(a, b):
    return lax.dot_general(a.astype(BF16), b.astype(BF16),
                           (((1,), (1,)), ((), ())),
                           preferred_element_type=F32)


def _dotb(a, b):
    return jnp.dot(a.astype(BF16), b.astype(BF16),
                   preferred_element_type=F32)


def _proj_body(x_ref, w_ref, b_ref, o_ref):
    o_ref[...] = _dotb(x_ref[...], w_ref[...]) + b_ref[...]


def _blk_body(seg, kf_ref, pos_ref, o_ref):
    for i in range(CBS):
        li = kf_ref[(i // CST):(i // CST) + MP,
                    (i % CST) * seg:((i % CST) + 1) * seg]
        o_ref[:, i * seg:(i + 1) * seg] = (
            li + pos_ref[:, i * seg:(i + 1) * seg]).astype(BF16)


def _cmp1_body(bl_ref, w1_ref, b1_ref, o_ref):
    o_ref[...] = _gelu(jnp.dot(bl_ref[...], w1_ref[...],
                               preferred_element_type=F32) + b1_ref[...])


def _cmp2_body(h_ref, w2_ref, b2_ref, o_ref):
    o_ref[...] = _dotb(h_ref[...], w2_ref[...]) + b2_ref[...]


def _catt_body(qc_ref, qs_ref, ck_ref, cv_ref, ocmp_ref, obm_ref):
    maddrow = jnp.where(
        lax.broadcasted_iota(jnp.int32, (1, MP), 1) < M, 0.0, NEG)
    imp = jnp.zeros((QT, MP), F32)
    for h in range(H):
        ckh = ck_ref[:, h * KD:(h + 1) * KD]
        s = _dot_nt(qc_ref[:, h * KD:(h + 1) * KD], ckh) * SCALE + maddrow
        mm = jnp.max(s, axis=1, keepdims=True)
        el = jnp.exp(s - mm)
        inv = 1.0 / jnp.sum(el, axis=1, keepdims=True)
        ocmp_ref[:, h * D:(h + 1) * D] = _dotb(
            el, cv_ref[:, h * D:(h + 1) * D]) * inv
        s2 = _dot_nt(qs_ref[:, h * KD:(h + 1) * KD], ckh) * SCALE + maddrow
        mm2 = jnp.max(s2, axis=1, keepdims=True)
        el2 = jnp.exp(s2 - mm2)
        inv2 = 1.0 / jnp.sum(el2, axis=1, keepdims=True)
        imp = imp + el2 * inv2
    impm = imp + maddrow
    vals = impm
    for _ in range(TOPN - 1):
        mx = jnp.max(vals, axis=1, keepdims=True)
        vals = jnp.where(vals >= mx, NEG, vals)
    thr = jnp.max(vals, axis=1, keepdims=True)
    shifted = jnp.concatenate(
        [jnp.full((QT, 1), NEG, F32), impm[:, :MP - 1]], axis=1)
    impm2 = jnp.maximum(impm, shifted)
    obm_ref[...] = (impm2 >= thr).astype(F32)


def _slc_body(nc, qoff, bm_ref, q_ref, k_ref, v_ref, o_ref,
              madd_scr, sc0, sc1, pb0, pb1):
    qt = pl.program_id(0) + qoff
    rows = qt * QT + lax.broadcasted_iota(jnp.int32, (QT, QT), 0)
    prow = lax.broadcasted_iota(jnp.int32, (32, QT), 0)
    jcol = lax.broadcasted_iota(jnp.int32, (32, QT), 1)
    e = (jcol // 8 == prow).astype(F32)
    for kt in range(nc):
        km = _dotb(bm_ref[:, kt * 32:(kt + 1) * 32], e)
        cols = kt * QT + lax.broadcasted_iota(jnp.int32, (QT, QT), 1)
        keep = (cols <= rows) & (km > 0.5)
        madd_scr[:, kt * QT:(kt + 1) * QT] = jnp.where(keep, 0.0, NEG)

    for h in range(H):
        sc = sc0 if h % 2 == 0 else sc1
        pb = pb0 if h % 2 == 0 else pb1
        q = q_ref[:, h * KD:(h + 1) * KD]
        mxs = []
        for kt in range(nc):
            k = k_ref[kt * QT:(kt + 1) * QT, h * KD:(h + 1) * KD]
            s = _dot_nt(q, k) * SCALE + madd_scr[:, kt * QT:(kt + 1) * QT]
            sc[:, kt * QT:(kt + 1) * QT] = s
            mxs.append(jnp.max(s, axis=1, keepdims=True))
        while len(mxs) > 1:
            mxs = [jnp.maximum(mxs[i], mxs[i + 1]) if i + 1 < len(mxs)
                   else mxs[i] for i in range(0, len(mxs), 2)]
        m = mxs[0]
        l = jnp.zeros((QT, 1), F32)
        for kt in range(nc):
            el = jnp.exp(sc[:, kt * QT:(kt + 1) * QT] - m)
            pb[:, kt * QT:(kt + 1) * QT] = el.astype(BF16)
            l = l + jnp.sum(el, axis=1, keepdims=True)
        inv = jnp.where(m > -1e29, 1.0 / l, 0.0)
        pv = jnp.dot(pb[...], v_ref[:, h * D:(h + 1) * D].astype(BF16),
                     preferred_element_type=F32)
        o_ref[:, h * D:(h + 1) * D] = pv * inv


def _win_body(q_ref, k_ref, v_ref, o_ref):
    qt = pl.program_id(0)
    rows = qt * QT + lax.broadcasted_iota(jnp.int32, (QT, QT), 0)
    madds = []
    kts = []
    for dd in range(3):
        kt = qt - 2 + dd
        cols = kt * QT + lax.broadcasted_iota(jnp.int32, (QT, QT), 1)
        keep = (cols <= rows) & (cols > rows - WIN) & (kt >= 0)
        madds.append(jnp.where(keep, 0.0, NEG))
        kts.append(jnp.maximum(kt, 0))
    for h in range(H):
        q = q_ref[:, h * KD:(h + 1) * KD]
        ss = []
        for dd in range(3):
            k = k_ref[pl.ds(kts[dd] * QT, QT), h * KD:(h + 1) * KD]
            ss.append(_dot_nt(q, k) * SCALE_W + madds[dd])
        m = jnp.maximum(jnp.maximum(
            jnp.max(ss[0], axis=1, keepdims=True),
            jnp.max(ss[1], axis=1, keepdims=True)),
            jnp.max(ss[2], axis=1, keepdims=True))
        l = jnp.zeros((QT, 1), F32)
        els = []
        for dd in range(3):
            el = jnp.exp(ss[dd] - m)
            els.append(el)
            l = l + jnp.sum(el, axis=1, keepdims=True)
        inv = 1.0 / l
        acc = jnp.zeros((QT, D), F32)
        for dd in range(3):
            v = v_ref[pl.ds(kts[dd] * QT, QT), h * D:(h + 1) * D]
            acc = acc + jnp.dot(els[dd].astype(BF16), v.astype(BF16),
                                preferred_element_type=F32)
        o_ref[:, h * D:(h + 1) * D] = acc * inv


def _fin_body(x_ref, gw1_ref, gb1_ref, gw2_ref, gb2_ref,
              cmp_ref, slc_ref, win_ref, pw_ref, pb_ref, o_ref):
    gh = _gelu(_dotb(x_ref[...], gw1_ref[...]) + gb1_ref[...])
    g = _sigmoid(_dotb(gh, gw2_ref[...]) + gb2_ref[...])
    comb = (g[:, 0:1] * cmp_ref[...] + g[:, 1:2] * slc_ref[...]
            + g[:, 2:3] * win_ref[...])
    o_ref[...] = _dotb(comb, pw_ref[...]) + pb_ref[...]


def _full(shape, imap):
    return pl.BlockSpec(shape, imap)


def kernel(x, W_cmp, b_cmp, W_slc, b_slc, W_win, b_win,
           k_pos, k_W1, k_b1, k_W2, k_b2,
           v_pos, v_W1, v_b1, v_W2, v_b2,
           g_W1, g_b1, g_W2, g_b2, p_W, p_b):
    x2 = x[0]
    Wall = jnp.concatenate([W_cmp, W_slc, W_win], axis=1).astype(BF16)
    ball = jnp.concatenate([b_cmp, b_slc, b_win])[None, :]

    qkv = pl.pallas_call(
        _proj_body,
        grid=(6,),
        in_specs=[
            _full((N, DIM), lambda ct: (0, 0)),
            _full((DIM, 768), lambda ct: (0, ct)),
            _full((1, 768), lambda ct: (0, ct)),
        ],
        out_specs=_full((N, 768), lambda ct: (0, ct)),
        out_shape=jax.ShapeDtypeStruct((N, 3 * QKV), F32),
    )(x2, Wall, ball)

    kflat = qkv[:, KC:2 * KC]
    vflat = qkv[:, 2 * KC:QKV]
    kfr = jnp.pad(kflat, ((0, 8), (0, 0))).reshape(MP + 1, 8 * KC)
    vfr = jnp.pad(vflat, ((0, 8), (0, 0))).reshape(MP + 1, 8 * VC)
    kposf = k_pos.reshape(1, CBS * KC)
    vposf = v_pos.reshape(1, CBS * VC)

    kbl = pl.pallas_call(
        lambda *a: _blk_body(KC, *a),
        grid=(1,),
        in_specs=[
            _full((MP + 1, 8 * KC), lambda i: (0, 0)),
            _full((1, CBS * KC), lambda i: (0, 0)),
        ],
        out_specs=_full((MP, CBS * KC), lambda i: (0, 0)),
        out_shape=jax.ShapeDtypeStruct((MP, CBS * KC), BF16),
    )(kfr, kposf)

    vbl = pl.pallas_call(
        lambda *a: _blk_body(VC, *a),
        grid=(1,),
        in_specs=[
            _full((MP + 1, 8 * VC), lambda i: (0, 0)),
            _full((1, CBS * VC), lambda i: (0, 0)),
        ],
        out_specs=_full((MP, CBS * VC), lambda i: (0, 0)),
        out_shape=jax.ShapeDtypeStruct((MP, CBS * VC), BF16),
    )(vfr, vposf)

    hk = pl.pallas_call(
        _cmp1_body,
        grid=(1,),
        in_specs=[
            _full((MP, CBS * KC), lambda i: (0, 0)),
            _full((CBS * KC, 2 * KC), lambda i: (0, 0)),
            _full((1, 2 * KC), lambda i: (0, 0)),
        ],
        out_specs=_full((MP, 2 * KC), lambda i: (0, 0)),
        out_shape=jax.ShapeDtypeStruct((MP, 2 * KC), F32),
    )(kbl, k_W1.astype(BF16), k_b1[None, :])

    ck = pl.pallas_call(
        _cmp2_body,
        grid=(1,),
        in_specs=[
            _full((MP, 2 * KC), lambda i: (0, 0)),
            _full((2 * KC, KC), lambda i: (0, 0)),
            _full((1, KC), lambda i: (0, 0)),
        ],
        out_specs=_full((MP, KC), lambda i: (0, 0)),
        out_shape=jax.ShapeDtypeStruct((MP, KC), F32),
    )(hk, k_W2.astype(BF16), k_b2[None, :])

    hv = pl.pallas_call(
        _cmp1_body,
        grid=(3,),
        in_specs=[
            _full((MP, CBS * VC), lambda ct: (0, 0)),
            _full((CBS * VC, 512), lambda ct: (0, ct)),
            _full((1, 512), lambda ct: (0, ct)),
        ],
        out_specs=_full((MP, 512), lambda ct: (0, ct)),
        out_shape=jax.ShapeDtypeStruct((MP, 2 * VC), F32),
    )(vbl, v_W1.astype(BF16), v_b1[None, :])

    cv = pl.pallas_call(
        _cmp2_body,
        grid=(1,),
        in_specs=[
            _full((MP, 2 * VC), lambda i: (0, 0)),
            _full((2 * VC, VC), lambda i: (0, 0)),
            _full((1, VC), lambda i: (0, 0)),
        ],
        out_specs=_full((MP, VC), lambda i: (0, 0)),
        out_shape=jax.ShapeDtypeStruct((MP, VC), F32),
    )(hv, v_W2.astype(BF16), v_b2[None, :])

    out_cmp, bmask = pl.pallas_call(
        _catt_body,
        grid=(NT,),
        in_specs=[
            _full((QT, KC), lambda qt: (qt, 0)),
            _full((QT, KC), lambda qt: (qt, 4)),
            _full((MP, KC), lambda qt: (0, 0)),
            _full((MP, VC), lambda qt: (0, 0)),
        ],
        out_specs=[
            _full((QT, VC), lambda qt: (qt, 0)),
            _full((QT, MP), lambda qt: (qt, 0)),
        ],
        out_shape=[
            jax.ShapeDtypeStruct((N, VC), F32),
            jax.ShapeDtypeStruct((N, MP), F32),
        ],
    )(qkv, qkv, ck, cv)

    def _slc_call(nc, qoff, nq):
        return pl.pallas_call(
            lambda *a: _slc_body(nc, qoff, *a),
            grid=(nq,),
            in_specs=[
                _full((QT, MP), lambda qt: (qt + qoff, 0)),
                _full((QT, KC), lambda qt: (qt + qoff, 4)),
                _full((nc * QT, KC), lambda qt: (0, 5)),
                _full((nc * QT, VC), lambda qt: (0, 3)),
            ],
            out_specs=_full((QT, VC), lambda qt: (qt, 0)),
            out_shape=jax.ShapeDtypeStruct((nq * QT, VC), F32),
            scratch_shapes=[
                pltpu.VMEM((QT, nc * QT), F32),
                pltpu.VMEM((QT, nc * QT), F32),
                pltpu.VMEM((QT, nc * QT), F32),
                pltpu.VMEM((QT, nc * QT), BF16),
                pltpu.VMEM((QT, nc * QT), BF16),
            ],
        )(bmask, qkv, qkv, qkv)

    out_slc = jnp.concatenate(
        [_slc_call(4, 0, 4), _slc_call(8, 4, 4)], axis=0)

    out_win = pl.pallas_call(
        _win_body,
        grid=(NT,),
        in_specs=[
            _full((QT, KC), lambda qt: (qt, 8)),
            _full((N, KC), lambda qt: (0, 9)),
            _full((N, VC), lambda qt: (0, 5)),
        ],
        out_specs=_full((QT, VC), lambda qt: (qt, 0)),
        out_shape=jax.ShapeDtypeStruct((N, VC), F32),
    )(qkv, qkv, qkv)

    gW2p = jnp.pad(g_W2, ((0, 0), (0, 125)))
    gb2p = jnp.pad(g_b2, (0, 125))[None, :]
    out = pl.pallas_call(
        _fin_body,
        grid=(NT,),
        in_specs=[
            _full((QT, DIM), lambda qt: (qt, 0)),
            _full((DIM, DIM // 2), lambda qt: (0, 0)),
            _full((1, DIM // 2), lambda qt: (0, 0)),
            _full((DIM // 2, 128), lambda qt: (0, 0)),
            _full((1, 128), lambda qt: (0, 0)),
            _full((QT, VC), lambda qt: (qt, 0)),
            _full((QT, VC), lambda qt: (qt, 0)),
            _full((QT, VC), lambda qt: (qt, 0)),
            _full((VC, DIM), lambda qt: (0, 0)),
            _full((1, DIM), lambda qt: (0, 0)),
        ],
        out_specs=_full((QT, DIM), lambda qt: (qt, 0)),
        out_shape=jax.ShapeDtypeStruct((N, DIM), F32),
    )(x2, g_W1.astype(BF16), g_b1[None, :], gW2p.astype(BF16), gb2p,
      out_cmp, out_slc, out_win, p_W.astype(BF16), p_b[None, :])

    return out[None, :, :]


# speedup vs baseline: 1.2930x; 1.0774x over previous
import jax
import jax.numpy as jnp
from jax import lax
from jax.experimental import pallas as pl
from jax.experimental.pallas import tpu as pltpu

N = 2048
DIM = 768
H = 12
KD = 32
D = 64
QKV = H * KD * 2 + H * D
CBS = 16
CST = 8
TOPN = 16
WIN = 512
KC = H * KD
VC = H * D
M = (N - CBS) // CST + 1
MP = 256
QT = 256
NT = N // QT
SCALE = KD ** (-0.5)
SCALE_W = (DIM // H) ** (-0.5)
F32 = jnp.float32
BF16 = jnp.bfloat16
NEG = -1e30


def _gelu(x):
    return 0.5 * x * (1.0 + lax.erf(x * 0.7071067811865476))


def _sigmoid(x):
    return 1.0 / (1.0 + jnp.exp(-x))


def _dot_nt(a, b):
    return lax.dot_general(a.astype(BF16), b.astype(BF16),
                           (((1,), (1,)), ((), ())),
                           preferred_element_type=F32)


def _dotb(a, b):
    return jnp.dot(a.astype(BF16), b.astype(BF16),
                   preferred_element_type=F32)


def _proj_body(x_ref, w_ref, b_ref, o_ref):
    o_ref[...] = _dotb(x_ref[...], w_ref[...]) + b_ref[...]


def _blk_body(seg, kf_ref, pos_ref, o_ref):
    for i in range(CBS):
        li = kf_ref[(i // CST):(i // CST) + MP,
                    (i % CST) * seg:((i % CST) + 1) * seg]
        o_ref[:, i * seg:(i + 1) * seg] = (
            li + pos_ref[:, i * seg:(i + 1) * seg]).astype(BF16)


def _cmp1_body(bl_ref, w1_ref, b1_ref, o_ref):
    o_ref[...] = _gelu(_dotb(bl_ref[...], w1_ref[...]) + b1_ref[...])


def _cmp2_body(h_ref, w2_ref, b2_ref, o_ref):
    o_ref[...] = _dotb(h_ref[...], w2_ref[...]) + b2_ref[...]


def _catt_body(qc_ref, qs_ref, ck_ref, cv_ref, ocmp_ref, obm_ref):
    maddrow = jnp.where(
        lax.broadcasted_iota(jnp.int32, (1, MP), 1) < M, 0.0, NEG)
    imp = jnp.zeros((QT, MP), F32)
    for h in range(H):
        ckh = ck_ref[:, h * KD:(h + 1) * KD]
        s = _dot_nt(qc_ref[:, h * KD:(h + 1) * KD], ckh) * SCALE + maddrow
        mm = jnp.max(s, axis=1, keepdims=True)
        el = jnp.exp(s - mm)
        inv = 1.0 / jnp.sum(el, axis=1, keepdims=True)
        ocmp_ref[:, h * D:(h + 1) * D] = _dotb(
            el, cv_ref[:, h * D:(h + 1) * D]) * inv
        s2 = _dot_nt(qs_ref[:, h * KD:(h + 1) * KD], ckh) * SCALE + maddrow
        mm2 = jnp.max(s2, axis=1, keepdims=True)
        el2 = jnp.exp(s2 - mm2)
        inv2 = 1.0 / jnp.sum(el2, axis=1, keepdims=True)
        imp = imp + el2 * inv2
    impm = imp + maddrow
    vals = impm
    for _ in range(TOPN - 1):
        mx = jnp.max(vals, axis=1, keepdims=True)
        vals = jnp.where(vals >= mx, NEG, vals)
    thr = jnp.max(vals, axis=1, keepdims=True)
    shifted = jnp.concatenate(
        [jnp.full((QT, 1), NEG, F32), impm[:, :MP - 1]], axis=1)
    impm2 = jnp.maximum(impm, shifted)
    obm_ref[...] = (impm2 >= thr).astype(F32)


def _slc_body(nc, qoff, bm_ref, q_ref, k_ref, v_ref, o_ref,
              madd_scr, sc0, sc1, pb0, pb1):
    qt = pl.program_id(0) + qoff
    rows = qt * QT + lax.broadcasted_iota(jnp.int32, (QT, QT), 0)
    prow = lax.broadcasted_iota(jnp.int32, (32, QT), 0)
    jcol = lax.broadcasted_iota(jnp.int32, (32, QT), 1)
    e = (jcol // 8 == prow).astype(F32)
    for kt in range(nc):
        km = _dotb(bm_ref[:, kt * 32:(kt + 1) * 32], e)
        cols = kt * QT + lax.broadcasted_iota(jnp.int32, (QT, QT), 1)
        keep = (cols <= rows) & (km > 0.5)
        madd_scr[:, kt * QT:(kt + 1) * QT] = jnp.where(keep, 0.0, NEG)

    for h in range(H):
        sc = sc0 if h % 2 == 0 else sc1
        pb = pb0 if h % 2 == 0 else pb1
        q = q_ref[:, h * KD:(h + 1) * KD]
        mxs = []
        for kt in range(nc):
            k = k_ref[kt * QT:(kt + 1) * QT, h * KD:(h + 1) * KD]
            s = _dot_nt(q, k) * SCALE + madd_scr[:, kt * QT:(kt + 1) * QT]
            sc[:, kt * QT:(kt + 1) * QT] = s
            mxs.append(jnp.max(s, axis=1, keepdims=True))
        while len(mxs) > 1:
            mxs = [jnp.maximum(mxs[i], mxs[i + 1]) if i + 1 < len(mxs)
                   else mxs[i] for i in range(0, len(mxs), 2)]
        m = mxs[0]
        l = jnp.zeros((QT, 1), F32)
        for kt in range(nc):
            el = jnp.exp(sc[:, kt * QT:(kt + 1) * QT] - m)
            pb[:, kt * QT:(kt + 1) * QT] = el.astype(BF16)
            l = l + jnp.sum(el, axis=1, keepdims=True)
        inv = jnp.where(m > -1e29, 1.0 / l, 0.0)
        pv = jnp.dot(pb[...], v_ref[:, h * D:(h + 1) * D].astype(BF16),
                     preferred_element_type=F32)
        o_ref[:, h * D:(h + 1) * D] = pv * inv


def _win_body(q_ref, k_ref, v_ref, o_ref):
    qt = pl.program_id(0)
    rows = qt * QT + lax.broadcasted_iota(jnp.int32, (QT, QT), 0)
    madds = []
    kts = []
    for dd in range(3):
        kt = qt - 2 + dd
        cols = kt * QT + lax.broadcasted_iota(jnp.int32, (QT, QT), 1)
        keep = (cols <= rows) & (cols > rows - WIN) & (kt >= 0)
        madds.append(jnp.where(keep, 0.0, NEG))
        kts.append(jnp.maximum(kt, 0))
    for h in range(H):
        q = q_ref[:, h * KD:(h + 1) * KD]
        ss = []
        for dd in range(3):
            k = k_ref[pl.ds(kts[dd] * QT, QT), h * KD:(h + 1) * KD]
            ss.append(_dot_nt(q, k) * SCALE_W + madds[dd])
        m = jnp.maximum(jnp.maximum(
            jnp.max(ss[0], axis=1, keepdims=True),
            jnp.max(ss[1], axis=1, keepdims=True)),
            jnp.max(ss[2], axis=1, keepdims=True))
        l = jnp.zeros((QT, 1), F32)
        els = []
        for dd in range(3):
            el = jnp.exp(ss[dd] - m)
            els.append(el)
            l = l + jnp.sum(el, axis=1, keepdims=True)
        inv = 1.0 / l
        acc = jnp.zeros((QT, D), F32)
        for dd in range(3):
            v = v_ref[pl.ds(kts[dd] * QT, QT), h * D:(h + 1) * D]
            acc = acc + jnp.dot(els[dd].astype(BF16), v.astype(BF16),
                                preferred_element_type=F32)
        o_ref[:, h * D:(h + 1) * D] = acc * inv


def _fin_body(x_ref, gw1_ref, gb1_ref, gw2_ref, gb2_ref,
              cmp_ref, slc_ref, win_ref, pw_ref, pb_ref, o_ref):
    gh = _gelu(_dotb(x_ref[...], gw1_ref[...]) + gb1_ref[...])
    g = _sigmoid(_dotb(gh, gw2_ref[...]) + gb2_ref[...])
    comb = (g[:, 0:1] * cmp_ref[...] + g[:, 1:2] * slc_ref[...]
            + g[:, 2:3] * win_ref[...])
    o_ref[...] = _dotb(comb, pw_ref[...]) + pb_ref[...]


def _full(shape, imap):
    return pl.BlockSpec(shape, imap)


def kernel(x, W_cmp, b_cmp, W_slc, b_slc, W_win, b_win,
           k_pos, k_W1, k_b1, k_W2, k_b2,
           v_pos, v_W1, v_b1, v_W2, v_b2,
           g_W1, g_b1, g_W2, g_b2, p_W, p_b):
    x2 = x[0]
    Wall = jnp.concatenate([W_cmp, W_slc, W_win], axis=1)
    ball = jnp.concatenate([b_cmp, b_slc, b_win])[None, :]

    qkv = pl.pallas_call(
        _proj_body,
        grid=(6,),
        in_specs=[
            _full((N, DIM), lambda ct: (0, 0)),
            _full((DIM, 768), lambda ct: (0, ct)),
            _full((1, 768), lambda ct: (0, ct)),
        ],
        out_specs=_full((N, 768), lambda ct: (0, ct)),
        out_shape=jax.ShapeDtypeStruct((N, 3 * QKV), F32),
    )(x2, Wall, ball)

    kflat = qkv[:, KC:2 * KC]
    vflat = qkv[:, 2 * KC:QKV]
    kfr = jnp.pad(kflat, ((0, 8), (0, 0))).reshape(MP + 1, 8 * KC)
    vfr = jnp.pad(vflat, ((0, 8), (0, 0))).reshape(MP + 1, 8 * VC)
    kposf = k_pos.reshape(1, CBS * KC)
    vposf = v_pos.reshape(1, CBS * VC)

    kbl = pl.pallas_call(
        lambda *a: _blk_body(KC, *a),
        grid=(1,),
        in_specs=[
            _full((MP + 1, 8 * KC), lambda i: (0, 0)),
            _full((1, CBS * KC), lambda i: (0, 0)),
        ],
        out_specs=_full((MP, CBS * KC), lambda i: (0, 0)),
        out_shape=jax.ShapeDtypeStruct((MP, CBS * KC), BF16),
    )(kfr, kposf)

    vbl = pl.pallas_call(
        lambda *a: _blk_body(VC, *a),
        grid=(1,),
        in_specs=[
            _full((MP + 1, 8 * VC), lambda i: (0, 0)),
            _full((1, CBS * VC), lambda i: (0, 0)),
        ],
        out_specs=_full((MP, CBS * VC), lambda i: (0, 0)),
        out_shape=jax.ShapeDtypeStruct((MP, CBS * VC), BF16),
    )(vfr, vposf)

    hk = pl.pallas_call(
        _cmp1_body,
        grid=(1,),
        in_specs=[
            _full((MP, CBS * KC), lambda i: (0, 0)),
            _full((CBS * KC, 2 * KC), lambda i: (0, 0)),
            _full((1, 2 * KC), lambda i: (0, 0)),
        ],
        out_specs=_full((MP, 2 * KC), lambda i: (0, 0)),
        out_shape=jax.ShapeDtypeStruct((MP, 2 * KC), F32),
    )(kbl, k_W1, k_b1[None, :])

    ck = pl.pallas_call(
        _cmp2_body,
        grid=(1,),
        in_specs=[
            _full((MP, 2 * KC), lambda i: (0, 0)),
            _full((2 * KC, KC), lambda i: (0, 0)),
            _full((1, KC), lambda i: (0, 0)),
        ],
        out_specs=_full((MP, KC), lambda i: (0, 0)),
        out_shape=jax.ShapeDtypeStruct((MP, KC), F32),
    )(hk, k_W2, k_b2[None, :])

    hv = pl.pallas_call(
        _cmp1_body,
        grid=(6,),
        in_specs=[
            _full((MP, CBS * VC), lambda ct: (0, 0)),
            _full((CBS * VC, 256), lambda ct: (0, ct)),
            _full((1, 256), lambda ct: (0, ct)),
        ],
        out_specs=_full((MP, 256), lambda ct: (0, ct)),
        out_shape=jax.ShapeDtypeStruct((MP, 2 * VC), F32),
    )(vbl, v_W1, v_b1[None, :])

    cv = pl.pallas_call(
        _cmp2_body,
        grid=(1,),
        in_specs=[
            _full((MP, 2 * VC), lambda i: (0, 0)),
            _full((2 * VC, VC), lambda i: (0, 0)),
            _full((1, VC), lambda i: (0, 0)),
        ],
        out_specs=_full((MP, VC), lambda i: (0, 0)),
        out_shape=jax.ShapeDtypeStruct((MP, VC), F32),
    )(hv, v_W2, v_b2[None, :])

    out_cmp, bmask = pl.pallas_call(
        _catt_body,
        grid=(NT,),
        in_specs=[
            _full((QT, KC), lambda qt: (qt, 0)),
            _full((QT, KC), lambda qt: (qt, 4)),
            _full((MP, KC), lambda qt: (0, 0)),
            _full((MP, VC), lambda qt: (0, 0)),
        ],
        out_specs=[
            _full((QT, VC), lambda qt: (qt, 0)),
            _full((QT, MP), lambda qt: (qt, 0)),
        ],
        out_shape=[
            jax.ShapeDtypeStruct((N, VC), F32),
            jax.ShapeDtypeStruct((N, MP), F32),
        ],
    )(qkv, qkv, ck, cv)

    def _slc_call(nc, qoff, nq):
        return pl.pallas_call(
            lambda *a: _slc_body(nc, qoff, *a),
            grid=(nq,),
            in_specs=[
                _full((QT, MP), lambda qt: (qt + qoff, 0)),
                _full((QT, KC), lambda qt: (qt + qoff, 4)),
                _full((nc * QT, KC), lambda qt: (0, 5)),
                _full((nc * QT, VC), lambda qt: (0, 3)),
            ],
            out_specs=_full((QT, VC), lambda qt: (qt, 0)),
            out_shape=jax.ShapeDtypeStruct((nq * QT, VC), F32),
            scratch_shapes=[
                pltpu.VMEM((QT, nc * QT), F32),
                pltpu.VMEM((QT, nc * QT), F32),
                pltpu.VMEM((QT, nc * QT), F32),
                pltpu.VMEM((QT, nc * QT), BF16),
                pltpu.VMEM((QT, nc * QT), BF16),
            ],
        )(bmask, qkv, qkv, qkv)

    out_slc = jnp.concatenate(
        [_slc_call(4, 0, 4), _slc_call(8, 4, 4)], axis=0)

    out_win = pl.pallas_call(
        _win_body,
        grid=(NT,),
        in_specs=[
            _full((QT, KC), lambda qt: (qt, 8)),
            _full((N, KC), lambda qt: (0, 9)),
            _full((N, VC), lambda qt: (0, 5)),
        ],
        out_specs=_full((QT, VC), lambda qt: (qt, 0)),
        out_shape=jax.ShapeDtypeStruct((N, VC), F32),
    )(qkv, qkv, qkv)

    gW2p = jnp.pad(g_W2, ((0, 0), (0, 125)))
    gb2p = jnp.pad(g_b2, (0, 125))[None, :]
    out = pl.pallas_call(
        _fin_body,
        grid=(NT,),
        in_specs=[
            _full((QT, DIM), lambda qt: (qt, 0)),
            _full((DIM, DIM // 2), lambda qt: (0, 0)),
            _full((1, DIM // 2), lambda qt: (0, 0)),
            _full((DIM // 2, 128), lambda qt: (0, 0)),
            _full((1, 128), lambda qt: (0, 0)),
            _full((QT, VC), lambda qt: (qt, 0)),
            _full((QT, VC), lambda qt: (qt, 0)),
            _full((QT, VC), lambda qt: (qt, 0)),
            _full((VC, DIM), lambda qt: (0, 0)),
            _full((1, DIM), lambda qt: (0, 0)),
        ],
        out_specs=_full((QT, DIM), lambda qt: (qt, 0)),
        out_shape=jax.ShapeDtypeStruct((N, DIM), F32),
    )(x2, g_W1, g_b1[None, :], gW2p, gb2p,
      out_cmp, out_slc, out_win, p_W, p_b[None, :])

    return out[None, :, :]


# speedup vs baseline: 1.3562x; 1.0489x over previous
import jax
import jax.numpy as jnp
from jax import lax
from jax.experimental import pallas as pl
from jax.experimental.pallas import tpu as pltpu

N = 2048
DIM = 768
H = 12
KD = 32
D = 64
QKV = H * KD * 2 + H * D
CBS = 16
CST = 8
TOPN = 16
WIN = 512
KC = H * KD
VC = H * D
M = (N - CBS) // CST + 1
MP = 256
QT = 256
NT = N // QT
SCALE = KD ** (-0.5)
SCALE_W = (DIM // H) ** (-0.5)
F32 = jnp.float32
BF16 = jnp.bfloat16
NEG = -1e30


def _gelu(x):
    return 0.5 * x * (1.0 + lax.erf(x * 0.7071067811865476))


def _sigmoid(x):
    return 1.0 / (1.0 + jnp.exp(-x))


def _dot_nt(a, b):
    return lax.dot_general(a.astype(BF16), b.astype(BF16),
                           (((1,), (1,)), ((), ())),
                           preferred_element_type=F32)


def _dotb(a, b):
    return jnp.dot(a.astype(BF16), b.astype(BF16),
                   preferred_element_type=F32)


def _proj_body(x_ref, w_ref, b_ref, o_ref):
    o_ref[...] = _dotb(x_ref[...], w_ref[...]) + b_ref[...]


def _blk_body(seg, kf_ref, pos_ref, o_ref):
    for i in range(CBS):
        li = kf_ref[(i // CST):(i // CST) + MP,
                    (i % CST) * seg:((i % CST) + 1) * seg]
        o_ref[:, i * seg:(i + 1) * seg] = (
            li + pos_ref[:, i * seg:(i + 1) * seg]).astype(BF16)


def _cmp1_body(bl_ref, w1_ref, b1_ref, o_ref):
    o_ref[...] = _gelu(_dotb(bl_ref[...], w1_ref[...]) + b1_ref[...])


def _cmp2_body(h_ref, w2_ref, b2_ref, o_ref):
    o_ref[...] = _dotb(h_ref[...], w2_ref[...]) + b2_ref[...]


def _catt_body(qc_ref, qs_ref, ck_ref, cv_ref, ocmp_ref, obm_ref):
    maddrow = jnp.where(
        lax.broadcasted_iota(jnp.int32, (1, MP), 1) < M, 0.0, NEG)
    imp = jnp.zeros((QT, MP), F32)
    for h in range(H):
        ckh = ck_ref[:, h * KD:(h + 1) * KD]
        s = _dot_nt(qc_ref[:, h * KD:(h + 1) * KD], ckh) * SCALE + maddrow
        mm = jnp.max(s, axis=1, keepdims=True)
        el = jnp.exp(s - mm)
        inv = 1.0 / jnp.sum(el, axis=1, keepdims=True)
        ocmp_ref[:, h * D:(h + 1) * D] = _dotb(
            el, cv_ref[:, h * D:(h + 1) * D]) * inv
        s2 = _dot_nt(qs_ref[:, h * KD:(h + 1) * KD], ckh) * SCALE + maddrow
        mm2 = jnp.max(s2, axis=1, keepdims=True)
        el2 = jnp.exp(s2 - mm2)
        inv2 = 1.0 / jnp.sum(el2, axis=1, keepdims=True)
        imp = imp + el2 * inv2
    impm = imp + maddrow
    vals = impm
    for _ in range(TOPN - 1):
        mx = jnp.max(vals, axis=1, keepdims=True)
        vals = jnp.where(vals >= mx, NEG, vals)
    thr = jnp.max(vals, axis=1, keepdims=True)
    shifted = jnp.concatenate(
        [jnp.full((QT, 1), NEG, F32), impm[:, :MP - 1]], axis=1)
    impm2 = jnp.maximum(impm, shifted)
    obm_ref[...] = (impm2 >= thr).astype(F32)


def _tail_body(nc, qoff, bm_ref, q_ref, k_ref, v_ref,
               qw_ref, kw_ref, vw_ref, x_ref, cmp_ref,
               gw1_ref, gb1_ref, gw2_ref, gb2_ref, pw_ref, pb_ref,
               o_ref, madd_scr, sc0, sc1, pb0, pb1, sacc, wacc):
    qt = pl.program_id(0) + qoff
    rows = qt * QT + lax.broadcasted_iota(jnp.int32, (QT, QT), 0)
    prow = lax.broadcasted_iota(jnp.int32, (32, QT), 0)
    jcol = lax.broadcasted_iota(jnp.int32, (32, QT), 1)
    e = (jcol // 8 == prow).astype(F32)
    for kt in range(nc):
        km = _dotb(bm_ref[:, kt * 32:(kt + 1) * 32], e)
        cols = kt * QT + lax.broadcasted_iota(jnp.int32, (QT, QT), 1)
        keep = (cols <= rows) & (km > 0.5)
        madd_scr[:, kt * QT:(kt + 1) * QT] = jnp.where(keep, 0.0, NEG)

    for h in range(H):
        sc = sc0 if h % 2 == 0 else sc1
        pb = pb0 if h % 2 == 0 else pb1
        q = q_ref[:, h * KD:(h + 1) * KD]
        mxs = []
        for kt in range(nc):
            k = k_ref[kt * QT:(kt + 1) * QT, h * KD:(h + 1) * KD]
            ss = _dot_nt(q, k) * SCALE + madd_scr[:, kt * QT:(kt + 1) * QT]
            sc[:, kt * QT:(kt + 1) * QT] = ss
            mxs.append(jnp.max(ss, axis=1, keepdims=True))
        while len(mxs) > 1:
            mxs = [jnp.maximum(mxs[i], mxs[i + 1]) if i + 1 < len(mxs)
                   else mxs[i] for i in range(0, len(mxs), 2)]
        m = mxs[0]
        l = jnp.zeros((QT, 1), F32)
        for kt in range(nc):
            el = jnp.exp(sc[:, kt * QT:(kt + 1) * QT] - m)
            pb[:, kt * QT:(kt + 1) * QT] = el.astype(BF16)
            l = l + jnp.sum(el, axis=1, keepdims=True)
        inv = jnp.where(m > -1e29, 1.0 / l, 0.0)
        pv = jnp.dot(pb[...], v_ref[:, h * D:(h + 1) * D].astype(BF16),
                     preferred_element_type=F32)
        sacc[:, h * D:(h + 1) * D] = pv * inv

    madds = []
    kts = []
    for dd in range(3):
        kt = qt - 2 + dd
        cols = kt * QT + lax.broadcasted_iota(jnp.int32, (QT, QT), 1)
        keep = (cols <= rows) & (cols > rows - WIN) & (kt >= 0)
        madds.append(jnp.where(keep, 0.0, NEG))
        kts.append(jnp.maximum(kt, 0))
    for h in range(H):
        q = qw_ref[:, h * KD:(h + 1) * KD]
        ss = []
        for dd in range(3):
            k = kw_ref[pl.ds(kts[dd] * QT, QT), h * KD:(h + 1) * KD]
            ss.append(_dot_nt(q, k) * SCALE_W + madds[dd])
        m = jnp.maximum(jnp.maximum(
            jnp.max(ss[0], axis=1, keepdims=True),
            jnp.max(ss[1], axis=1, keepdims=True)),
            jnp.max(ss[2], axis=1, keepdims=True))
        l = jnp.zeros((QT, 1), F32)
        els = []
        for dd in range(3):
            el = jnp.exp(ss[dd] - m)
            els.append(el)
            l = l + jnp.sum(el, axis=1, keepdims=True)
        inv = 1.0 / l
        acc = jnp.zeros((QT, D), F32)
        for dd in range(3):
            v = vw_ref[pl.ds(kts[dd] * QT, QT), h * D:(h + 1) * D]
            acc = acc + jnp.dot(els[dd].astype(BF16), v.astype(BF16),
                                preferred_element_type=F32)
        wacc[:, h * D:(h + 1) * D] = acc * inv

    gh = _gelu(_dotb(x_ref[...], gw1_ref[...]) + gb1_ref[...])
    g = _sigmoid(_dotb(gh, gw2_ref[...]) + gb2_ref[...])
    comb = (g[:, 0:1] * cmp_ref[...] + g[:, 1:2] * sacc[...]
            + g[:, 2:3] * wacc[...])
    o_ref[...] = _dotb(comb, pw_ref[...]) + pb_ref[...]


def _full(shape, imap):
    return pl.BlockSpec(shape, imap)


def kernel(x, W_cmp, b_cmp, W_slc, b_slc, W_win, b_win,
           k_pos, k_W1, k_b1, k_W2, k_b2,
           v_pos, v_W1, v_b1, v_W2, v_b2,
           g_W1, g_b1, g_W2, g_b2, p_W, p_b):
    x2 = x[0]
    Wall = jnp.concatenate([W_cmp, W_slc, W_win], axis=1)
    ball = jnp.concatenate([b_cmp, b_slc, b_win])[None, :]

    qkv = pl.pallas_call(
        _proj_body,
        grid=(6,),
        in_specs=[
            _full((N, DIM), lambda ct: (0, 0)),
            _full((DIM, 768), lambda ct: (0, ct)),
            _full((1, 768), lambda ct: (0, ct)),
        ],
        out_specs=_full((N, 768), lambda ct: (0, ct)),
        out_shape=jax.ShapeDtypeStruct((N, 3 * QKV), F32),
    )(x2, Wall, ball)

    kflat = qkv[:, KC:2 * KC]
    vflat = qkv[:, 2 * KC:QKV]
    kfr = jnp.pad(kflat, ((0, 8), (0, 0))).reshape(MP + 1, 8 * KC)
    vfr = jnp.pad(vflat, ((0, 8), (0, 0))).reshape(MP + 1, 8 * VC)
    kposf = k_pos.reshape(1, CBS * KC)
    vposf = v_pos.reshape(1, CBS * VC)

    kbl = pl.pallas_call(
        lambda *a: _blk_body(KC, *a),
        grid=(1,),
        in_specs=[
            _full((MP + 1, 8 * KC), lambda i: (0, 0)),
            _full((1, CBS * KC), lambda i: (0, 0)),
        ],
        out_specs=_full((MP, CBS * KC), lambda i: (0, 0)),
        out_shape=jax.ShapeDtypeStruct((MP, CBS * KC), BF16),
    )(kfr, kposf)

    vbl = pl.pallas_call(
        lambda *a: _blk_body(VC, *a),
        grid=(1,),
        in_specs=[
            _full((MP + 1, 8 * VC), lambda i: (0, 0)),
            _full((1, CBS * VC), lambda i: (0, 0)),
        ],
        out_specs=_full((MP, CBS * VC), lambda i: (0, 0)),
        out_shape=jax.ShapeDtypeStruct((MP, CBS * VC), BF16),
    )(vfr, vposf)

    hk = pl.pallas_call(
        _cmp1_body,
        grid=(1,),
        in_specs=[
            _full((MP, CBS * KC), lambda i: (0, 0)),
            _full((CBS * KC, 2 * KC), lambda i: (0, 0)),
            _full((1, 2 * KC), lambda i: (0, 0)),
        ],
        out_specs=_full((MP, 2 * KC), lambda i: (0, 0)),
        out_shape=jax.ShapeDtypeStruct((MP, 2 * KC), F32),
    )(kbl, k_W1, k_b1[None, :])

    ck = pl.pallas_call(
        _cmp2_body,
        grid=(1,),
        in_specs=[
            _full((MP, 2 * KC), lambda i: (0, 0)),
            _full((2 * KC, KC), lambda i: (0, 0)),
            _full((1, KC), lambda i: (0, 0)),
        ],
        out_specs=_full((MP, KC), lambda i: (0, 0)),
        out_shape=jax.ShapeDtypeStruct((MP, KC), F32),
    )(hk, k_W2, k_b2[None, :])

    hv = pl.pallas_call(
        _cmp1_body,
        grid=(6,),
        in_specs=[
            _full((MP, CBS * VC), lambda ct: (0, 0)),
            _full((CBS * VC, 256), lambda ct: (0, ct)),
            _full((1, 256), lambda ct: (0, ct)),
        ],
        out_specs=_full((MP, 256), lambda ct: (0, ct)),
        out_shape=jax.ShapeDtypeStruct((MP, 2 * VC), F32),
    )(vbl, v_W1, v_b1[None, :])

    cv = pl.pallas_call(
        _cmp2_body,
        grid=(1,),
        in_specs=[
            _full((MP, 2 * VC), lambda i: (0, 0)),
            _full((2 * VC, VC), lambda i: (0, 0)),
            _full((1, VC), lambda i: (0, 0)),
        ],
        out_specs=_full((MP, VC), lambda i: (0, 0)),
        out_shape=jax.ShapeDtypeStruct((MP, VC), F32),
    )(hv, v_W2, v_b2[None, :])

    out_cmp, bmask = pl.pallas_call(
        _catt_body,
        grid=(NT,),
        in_specs=[
            _full((QT, KC), lambda qt: (qt, 0)),
            _full((QT, KC), lambda qt: (qt, 4)),
            _full((MP, KC), lambda qt: (0, 0)),
            _full((MP, VC), lambda qt: (0, 0)),
        ],
        out_specs=[
            _full((QT, VC), lambda qt: (qt, 0)),
            _full((QT, MP), lambda qt: (qt, 0)),
        ],
        out_shape=[
            jax.ShapeDtypeStruct((N, VC), F32),
            jax.ShapeDtypeStruct((N, MP), F32),
        ],
    )(qkv, qkv, ck, cv)

    gW2p = jnp.pad(g_W2, ((0, 0), (0, 125)))
    gb2p = jnp.pad(g_b2, (0, 125))[None, :]

    def _tail_call(nc, qoff, nq):
        return pl.pallas_call(
            lambda *a: _tail_body(nc, qoff, *a),
            grid=(nq,),
            in_specs=[
                _full((QT, MP), lambda qt: (qt + qoff, 0)),
                _full((QT, KC), lambda qt: (qt + qoff, 4)),
                _full((nc * QT, KC), lambda qt: (0, 5)),
                _full((nc * QT, VC), lambda qt: (0, 3)),
                _full((QT, KC), lambda qt: (qt + qoff, 8)),
                _full((nc * QT, KC), lambda qt: (0, 9)),
                _full((nc * QT, VC), lambda qt: (0, 5)),
                _full((QT, DIM), lambda qt: (qt + qoff, 0)),
                _full((QT, VC), lambda qt: (qt + qoff, 0)),
                _full((DIM, DIM // 2), lambda qt: (0, 0)),
                _full((1, DIM // 2), lambda qt: (0, 0)),
                _full((DIM // 2, 128), lambda qt: (0, 0)),
                _full((1, 128), lambda qt: (0, 0)),
                _full((VC, DIM), lambda qt: (0, 0)),
                _full((1, DIM), lambda qt: (0, 0)),
            ],
            out_specs=_full((QT, DIM), lambda qt: (qt, 0)),
            out_shape=jax.ShapeDtypeStruct((nq * QT, DIM), F32),
            scratch_shapes=[
                pltpu.VMEM((QT, nc * QT), F32),
                pltpu.VMEM((QT, nc * QT), F32),
                pltpu.VMEM((QT, nc * QT), F32),
                pltpu.VMEM((QT, nc * QT), BF16),
                pltpu.VMEM((QT, nc * QT), BF16),
                pltpu.VMEM((QT, VC), F32),
                pltpu.VMEM((QT, VC), F32),
            ],
        )(bmask, qkv, qkv, qkv, qkv, qkv, qkv, x2, out_cmp,
          g_W1, g_b1[None, :], gW2p, gb2p, p_W, p_b[None, :])

    out = jnp.concatenate([_tail_call(4, 0, 4), _tail_call(8, 4, 4)], axis=0)

    return out[None, :, :]


# speedup vs baseline: 1.3615x; 1.0039x over previous
import jax
import jax.numpy as jnp
from jax import lax
from jax.experimental import pallas as pl
from jax.experimental.pallas import tpu as pltpu

N = 2048
DIM = 768
H = 12
KD = 32
D = 64
QKV = H * KD * 2 + H * D
CBS = 16
CST = 8
TOPN = 16
WIN = 512
KC = H * KD
VC = H * D
M = (N - CBS) // CST + 1
MP = 256
QT = 256
NT = N // QT
SCALE = KD ** (-0.5)
SCALE_W = (DIM // H) ** (-0.5)
F32 = jnp.float32
BF16 = jnp.bfloat16
NEG = -1e30


def _gelu(x):
    return 0.5 * x * (1.0 + lax.erf(x * 0.7071067811865476))


def _sigmoid(x):
    return 1.0 / (1.0 + jnp.exp(-x))


def _dot_nt(a, b):
    return lax.dot_general(a.astype(BF16), b.astype(BF16),
                           (((1,), (1,)), ((), ())),
                           preferred_element_type=F32)


def _dotb(a, b):
    return jnp.dot(a.astype(BF16), b.astype(BF16),
                   preferred_element_type=F32)


def _proj_body(x_ref, w_ref, b_ref, o_ref):
    o_ref[...] = _dotb(x_ref[...], w_ref[...]) + b_ref[...]


def _blk_body(seg, kf_ref, pos_ref, o_ref):
    for i in range(CBS):
        li = kf_ref[(i // CST):(i // CST) + MP,
                    (i % CST) * seg:((i % CST) + 1) * seg]
        o_ref[:, i * seg:(i + 1) * seg] = (
            li + pos_ref[:, i * seg:(i + 1) * seg]).astype(BF16)


def _cmp1_body(bl_ref, w1_ref, b1_ref, o_ref):
    o_ref[...] = _gelu(_dotb(bl_ref[...], w1_ref[...]) + b1_ref[...])


def _cmp2_body(h_ref, w2_ref, b2_ref, o_ref):
    o_ref[...] = _dotb(h_ref[...], w2_ref[...]) + b2_ref[...]


def _catt_body(qc_ref, qs_ref, ck_ref, cv_ref, ocmp_ref, obm_ref):
    maddrow = jnp.where(
        lax.broadcasted_iota(jnp.int32, (1, MP), 1) < M, 0.0, NEG)
    imp = jnp.zeros((QT, MP), F32)
    for h in range(H):
        ckh = ck_ref[:, h * KD:(h + 1) * KD]
        s = _dot_nt(qc_ref[:, h * KD:(h + 1) * KD], ckh) * SCALE + maddrow
        mm = jnp.max(s, axis=1, keepdims=True)
        el = jnp.exp(s - mm)
        inv = 1.0 / jnp.sum(el, axis=1, keepdims=True)
        ocmp_ref[:, h * D:(h + 1) * D] = _dotb(
            el, cv_ref[:, h * D:(h + 1) * D]) * inv
        s2 = _dot_nt(qs_ref[:, h * KD:(h + 1) * KD], ckh) * SCALE + maddrow
        mm2 = jnp.max(s2, axis=1, keepdims=True)
        el2 = jnp.exp(s2 - mm2)
        inv2 = 1.0 / jnp.sum(el2, axis=1, keepdims=True)
        imp = imp + el2 * inv2
    impm = imp + maddrow
    vals = impm
    for _ in range(TOPN - 1):
        mx = jnp.max(vals, axis=1, keepdims=True)
        vals = jnp.where(vals >= mx, NEG, vals)
    thr = jnp.max(vals, axis=1, keepdims=True)
    shifted = jnp.concatenate(
        [jnp.full((QT, 1), NEG, F32), impm[:, :MP - 1]], axis=1)
    impm2 = jnp.maximum(impm, shifted)
    obm_ref[...] = (impm2 >= thr).astype(F32)


def _tail_body(nc, qoff, bm_ref, q_ref, k_ref, v_ref,
               qw_ref, kw_ref, vw_ref, x_ref, cmp_ref,
               gw1_ref, gb1_ref, gw2_ref, gb2_ref, pw_ref, pb_ref,
               o_ref, madd_scr, sc0, sc1, pb0, pb1, sacc, wacc):
    qt = pl.program_id(0) + qoff
    rows = qt * QT + lax.broadcasted_iota(jnp.int32, (QT, QT), 0)
    prow = lax.broadcasted_iota(jnp.int32, (32, QT), 0)
    jcol = lax.broadcasted_iota(jnp.int32, (32, QT), 1)
    e = (jcol // 8 == prow).astype(F32)
    for kt in range(nc):
        km = _dotb(bm_ref[:, kt * 32:(kt + 1) * 32], e)
        cols = kt * QT + lax.broadcasted_iota(jnp.int32, (QT, QT), 1)
        keep = (cols <= rows) & (km > 0.5)
        madd_scr[:, kt * QT:(kt + 1) * QT] = jnp.where(keep, 0.0, NEG)

    for h in range(H):
        sc = sc0 if h % 2 == 0 else sc1
        pb = pb0 if h % 2 == 0 else pb1
        q = q_ref[:, h * KD:(h + 1) * KD]
        mxs = []
        for kt in range(nc):
            k = k_ref[kt * QT:(kt + 1) * QT, h * KD:(h + 1) * KD]
            ss = _dot_nt(q, k) * SCALE + madd_scr[:, kt * QT:(kt + 1) * QT]
            sc[:, kt * QT:(kt + 1) * QT] = ss
            mxs.append(jnp.max(ss, axis=1, keepdims=True))
        while len(mxs) > 1:
            mxs = [jnp.maximum(mxs[i], mxs[i + 1]) if i + 1 < len(mxs)
                   else mxs[i] for i in range(0, len(mxs), 2)]
        m = mxs[0]
        l = jnp.zeros((QT, 1), F32)
        for kt in range(nc):
            el = jnp.exp(sc[:, kt * QT:(kt + 1) * QT] - m)
            pb[:, kt * QT:(kt + 1) * QT] = el.astype(BF16)
            l = l + jnp.sum(el, axis=1, keepdims=True)
        inv = jnp.where(m > -1e29, 1.0 / l, 0.0)
        pv = jnp.dot(pb[...], v_ref[:, h * D:(h + 1) * D].astype(BF16),
                     preferred_element_type=F32)
        sacc[:, h * D:(h + 1) * D] = pv * inv

    madds = []
    kts = []
    for dd in range(3):
        kt = qt - 2 + dd
        cols = kt * QT + lax.broadcasted_iota(jnp.int32, (QT, QT), 1)
        keep = (cols <= rows) & (cols > rows - WIN) & (kt >= 0)
        madds.append(jnp.where(keep, 0.0, NEG))
        kts.append(jnp.maximum(kt, 0))
    for h in range(H):
        q = qw_ref[:, h * KD:(h + 1) * KD]
        ss = []
        for dd in range(3):
            k = kw_ref[pl.ds(kts[dd] * QT, QT), h * KD:(h + 1) * KD]
            ss.append(_dot_nt(q, k) * SCALE_W + madds[dd])
        m = jnp.maximum(jnp.maximum(
            jnp.max(ss[0], axis=1, keepdims=True),
            jnp.max(ss[1], axis=1, keepdims=True)),
            jnp.max(ss[2], axis=1, keepdims=True))
        l = jnp.zeros((QT, 1), F32)
        els = []
        for dd in range(3):
            el = jnp.exp(ss[dd] - m)
            els.append(el)
            l = l + jnp.sum(el, axis=1, keepdims=True)
        inv = 1.0 / l
        acc = jnp.zeros((QT, D), F32)
        for dd in range(3):
            v = vw_ref[pl.ds(kts[dd] * QT, QT), h * D:(h + 1) * D]
            acc = acc + jnp.dot(els[dd].astype(BF16), v.astype(BF16),
                                preferred_element_type=F32)
        wacc[:, h * D:(h + 1) * D] = acc * inv

    gh = _gelu(_dotb(x_ref[...], gw1_ref[...]) + gb1_ref[...])
    g = _sigmoid(_dotb(gh, gw2_ref[...]) + gb2_ref[...])
    comb = (g[:, 0:1] * cmp_ref[...] + g[:, 1:2] * sacc[...]
            + g[:, 2:3] * wacc[...])
    o_ref[...] = _dotb(comb, pw_ref[...]) + pb_ref[...]


def _full(shape, imap):
    return pl.BlockSpec(shape, imap)


def kernel(x, W_cmp, b_cmp, W_slc, b_slc, W_win, b_win,
           k_pos, k_W1, k_b1, k_W2, k_b2,
           v_pos, v_W1, v_b1, v_W2, v_b2,
           g_W1, g_b1, g_W2, g_b2, p_W, p_b):
    x2 = x[0]
    Wsw = jnp.concatenate([W_slc, W_win], axis=1)
    bsw = jnp.concatenate([b_slc, b_win])[None, :]

    qkvc = pl.pallas_call(
        _proj_body,
        grid=(2,),
        in_specs=[
            _full((N, DIM), lambda ct: (0, 0)),
            _full((DIM, 768), lambda ct: (0, ct)),
            _full((1, 768), lambda ct: (0, ct)),
        ],
        out_specs=_full((N, 768), lambda ct: (0, ct)),
        out_shape=jax.ShapeDtypeStruct((N, QKV), F32),
    )(x2, W_cmp, b_cmp[None, :])

    qsw = pl.pallas_call(
        lambda xr, wr, br, orf: orf.__setitem__(
            ..., (_dotb(xr[...], wr[...]) + br[...]).astype(BF16)),
        grid=(4,),
        in_specs=[
            _full((N, DIM), lambda ct: (0, 0)),
            _full((DIM, 768), lambda ct: (0, ct)),
            _full((1, 768), lambda ct: (0, ct)),
        ],
        out_specs=_full((N, 768), lambda ct: (0, ct)),
        out_shape=jax.ShapeDtypeStruct((N, 2 * QKV), BF16),
    )(x2, Wsw, bsw)

    kflat = qkvc[:, KC:2 * KC]
    vflat = qkvc[:, 2 * KC:QKV]
    kfr = jnp.pad(kflat, ((0, 8), (0, 0))).reshape(MP + 1, 8 * KC)
    vfr = jnp.pad(vflat, ((0, 8), (0, 0))).reshape(MP + 1, 8 * VC)
    kposf = k_pos.reshape(1, CBS * KC)
    vposf = v_pos.reshape(1, CBS * VC)

    kbl = pl.pallas_call(
        lambda *a: _blk_body(KC, *a),
        grid=(1,),
        in_specs=[
            _full((MP + 1, 8 * KC), lambda i: (0, 0)),
            _full((1, CBS * KC), lambda i: (0, 0)),
        ],
        out_specs=_full((MP, CBS * KC), lambda i: (0, 0)),
        out_shape=jax.ShapeDtypeStruct((MP, CBS * KC), BF16),
    )(kfr, kposf)

    vbl = pl.pallas_call(
        lambda *a: _blk_body(VC, *a),
        grid=(1,),
        in_specs=[
            _full((MP + 1, 8 * VC), lambda i: (0, 0)),
            _full((1, CBS * VC), lambda i: (0, 0)),
        ],
        out_specs=_full((MP, CBS * VC), lambda i: (0, 0)),
        out_shape=jax.ShapeDtypeStruct((MP, CBS * VC), BF16),
    )(vfr, vposf)

    hk = pl.pallas_call(
        _cmp1_body,
        grid=(1,),
        in_specs=[
            _full((MP, CBS * KC), lambda i: (0, 0)),
            _full((CBS * KC, 2 * KC), lambda i: (0, 0)),
            _full((1, 2 * KC), lambda i: (0, 0)),
        ],
        out_specs=_full((MP, 2 * KC), lambda i: (0, 0)),
        out_shape=jax.ShapeDtypeStruct((MP, 2 * KC), F32),
    )(kbl, k_W1, k_b1[None, :])

    ck = pl.pallas_call(
        _cmp2_body,
        grid=(1,),
        in_specs=[
            _full((MP, 2 * KC), lambda i: (0, 0)),
            _full((2 * KC, KC), lambda i: (0, 0)),
            _full((1, KC), lambda i: (0, 0)),
        ],
        out_specs=_full((MP, KC), lambda i: (0, 0)),
        out_shape=jax.ShapeDtypeStruct((MP, KC), F32),
    )(hk, k_W2, k_b2[None, :])

    hv = pl.pallas_call(
        _cmp1_body,
        grid=(6,),
        in_specs=[
            _full((MP, CBS * VC), lambda ct: (0, 0)),
            _full((CBS * VC, 256), lambda ct: (0, ct)),
            _full((1, 256), lambda ct: (0, ct)),
        ],
        out_specs=_full((MP, 256), lambda ct: (0, ct)),
        out_shape=jax.ShapeDtypeStruct((MP, 2 * VC), F32),
    )(vbl, v_W1, v_b1[None, :])

    cv = pl.pallas_call(
        _cmp2_body,
        grid=(1,),
        in_specs=[
            _full((MP, 2 * VC), lambda i: (0, 0)),
            _full((2 * VC, VC), lambda i: (0, 0)),
            _full((1, VC), lambda i: (0, 0)),
        ],
        out_specs=_full((MP, VC), lambda i: (0, 0)),
        out_shape=jax.ShapeDtypeStruct((MP, VC), F32),
    )(hv, v_W2, v_b2[None, :])

    out_cmp, bmask = pl.pallas_call(
        _catt_body,
        grid=(NT,),
        in_specs=[
            _full((QT, KC), lambda qt: (qt, 0)),
            _full((QT, KC), lambda qt: (qt, 0)),
            _full((MP, KC), lambda qt: (0, 0)),
            _full((MP, VC), lambda qt: (0, 0)),
        ],
        out_specs=[
            _full((QT, VC), lambda qt: (qt, 0)),
            _full((QT, MP), lambda qt: (qt, 0)),
        ],
        out_shape=[
            jax.ShapeDtypeStruct((N, VC), F32),
            jax.ShapeDtypeStruct((N, MP), F32),
        ],
    )(qkvc, qsw, ck, cv)

    gW2p = jnp.pad(g_W2, ((0, 0), (0, 125)))
    gb2p = jnp.pad(g_b2, (0, 125))[None, :]

    def _tail_call(nc, qoff, nq):
        return pl.pallas_call(
            lambda *a: _tail_body(nc, qoff, *a),
            grid=(nq,),
            in_specs=[
                _full((QT, MP), lambda qt: (qt + qoff, 0)),
                _full((QT, KC), lambda qt: (qt + qoff, 0)),
                _full((nc * QT, KC), lambda qt: (0, 1)),
                _full((nc * QT, VC), lambda qt: (0, 1)),
                _full((QT, KC), lambda qt: (qt + qoff, 4)),
                _full((nc * QT, KC), lambda qt: (0, 5)),
                _full((nc * QT, VC), lambda qt: (0, 3)),
                _full((QT, DIM), lambda qt: (qt + qoff, 0)),
                _full((QT, VC), lambda qt: (qt + qoff, 0)),
                _full((DIM, DIM // 2), lambda qt: (0, 0)),
                _full((1, DIM // 2), lambda qt: (0, 0)),
                _full((DIM // 2, 128), lambda qt: (0, 0)),
                _full((1, 128), lambda qt: (0, 0)),
                _full((VC, DIM), lambda qt: (0, 0)),
                _full((1, DIM), lambda qt: (0, 0)),
            ],
            out_specs=_full((QT, DIM), lambda qt: (qt, 0)),
            out_shape=jax.ShapeDtypeStruct((nq * QT, DIM), F32),
            scratch_shapes=[
                pltpu.VMEM((QT, nc * QT), F32),
                pltpu.VMEM((QT, nc * QT), F32),
                pltpu.VMEM((QT, nc * QT), F32),
                pltpu.VMEM((QT, nc * QT), BF16),
                pltpu.VMEM((QT, nc * QT), BF16),
                pltpu.VMEM((QT, VC), F32),
                pltpu.VMEM((QT, VC), F32),
            ],
        )(bmask, qsw, qsw, qsw, qsw, qsw, qsw, x2, out_cmp,
          g_W1, g_b1[None, :], gW2p, gb2p, p_W, p_b[None, :])

    out = jnp.concatenate([_tail_call(4, 0, 4), _tail_call(8, 4, 4)], axis=0)

    return out[None, :, :]
